# baseline (speedup 1.0000x reference)
import functools

import jax
import jax.numpy as jnp
from jax import lax
from jax.experimental import pallas as pl
from jax.experimental.pallas import tpu as pltpu
from jax.experimental.pallas import tpu_sc as plsc

N_NODES = 10000
D = 128
E = 320000

NC = 2
NS = 16
NW = NC * NS
K = 128
EPT = E // NW
NCH = 80
EPP = NCH * K
PAD = EPP - EPT
NPAD = N_NODES + 8
ZB = 1000
NZ = N_NODES // ZB
WB = 40
QD = 8

_MESH = plsc.VectorSubcoreMesh(core_axis_name="c", subcore_axis_name="s")


def _deg_body(col3_hbm, ones_hbm, zeros_hbm, degp_hbm, cidx_v, ones_v,
              stage_v, deg_sh, sem):
    cid = lax.axis_index("c")
    sid = lax.axis_index("s")
    wid = cid * NS + sid

    @pl.when(sid < NZ)
    def _():
        pltpu.sync_copy(zeros_hbm, stage_v)
        pltpu.sync_copy(stage_v, deg_sh.at[pl.ds(sid * ZB, ZB)])

    @pl.when(sid == NZ)
    def _():
        pltpu.sync_copy(zeros_hbm.at[pl.ds(0, NPAD - N_NODES)], stage_v.at[pl.ds(0, NPAD - N_NODES)])
        pltpu.sync_copy(stage_v.at[pl.ds(0, NPAD - N_NODES)],
                        deg_sh.at[pl.ds(N_NODES, NPAD - N_NODES)])

    pltpu.sync_copy(ones_hbm, ones_v)
    pltpu.sync_copy(col3_hbm.at[wid], cidx_v)
    plsc.subcore_barrier()

    def body(c, carry):
        pltpu.async_copy(ones_v, deg_sh.at[cidx_v.at[c]], sem, add=True)

        @pl.when(c >= QD)
        def _():
            pltpu.make_async_copy(ones_v, deg_sh.at[cidx_v.at[c - QD]],
                                  sem).wait()

        return carry

    lax.fori_loop(0, NCH, body, 0)

    def drain(c, carry):
        pltpu.make_async_copy(ones_v, deg_sh.at[cidx_v.at[c]], sem).wait()
        return carry

    lax.fori_loop(NCH - QD, NCH, drain, 0)

    plsc.subcore_barrier()

    @pl.when(sid < NZ)
    def _():
        pltpu.sync_copy(deg_sh.at[pl.ds(sid * ZB, ZB)], stage_v)
        pltpu.sync_copy(stage_v,
                        degp_hbm.at[pl.ds(cid * N_NODES + sid * ZB, ZB)])


_deg_call = pl.kernel(
    _deg_body,
    out_type=jax.ShapeDtypeStruct((NC * N_NODES,), jnp.float32),
    mesh=_MESH,
    scratch_types=[
        pltpu.VMEM((NCH, K), jnp.int32),
        pltpu.VMEM((K,), jnp.float32),
        pltpu.VMEM((ZB,), jnp.float32),
        pltpu.VMEM_SHARED((NPAD,), jnp.float32),
        pltpu.SemaphoreType.DMA,
    ],
)


def _acc_body(g_hbm, row3_hbm, colf_hbm, zrows_hbm, accp_hbm, ridx_v,
              cidx_a, cidx_b, rows_a, rows_b, acc_sh,
              gs_a, gs_b, ss_a, ss_b):
    cid = lax.axis_index("c")
    sid = lax.axis_index("s")
    wid = cid * NS + sid
    ebase = wid * EPP

    pltpu.sync_copy(zrows_hbm, rows_a.at[pl.ds(0, WB)])

    @pl.when(sid < NZ)
    def _():
        def zbody(j, carry):
            pltpu.sync_copy(rows_a.at[pl.ds(0, WB)],
                            acc_sh.at[pl.ds(sid * ZB + j * WB, WB)])
            return carry

        lax.fori_loop(0, ZB // WB, zbody, 0)

    @pl.when(sid == NZ)
    def _():
        pltpu.sync_copy(rows_a.at[pl.ds(0, NPAD - N_NODES)],
                        acc_sh.at[pl.ds(N_NODES, NPAD - N_NODES)])

    pltpu.sync_copy(row3_hbm.at[wid], ridx_v)
    plsc.subcore_barrier()

    def fire_gather(c, rows, sem):
        pltpu.async_copy(g_hbm.at[ridx_v.at[c]], rows, sem)

    def wait_gather(c, rows, sem):
        pltpu.make_async_copy(g_hbm.at[ridx_v.at[c]], rows, sem).wait()

    def load_col(c, cidx):
        pltpu.sync_copy(colf_hbm.at[pl.ds(ebase + c * K, K)], cidx)

    def fire_scatter(rows, cidx, sem):
        pltpu.async_copy(rows, acc_sh.at[cidx], sem, add=True)

    def wait_scatter(rows, cidx, sem):
        pltpu.make_async_copy(rows, acc_sh.at[cidx], sem).wait()

    load_col(0, cidx_a)
    fire_gather(0, rows_a, gs_a)
    load_col(1, cidx_b)
    wait_gather(0, rows_a, gs_a)
    fire_scatter(rows_a, cidx_a, ss_a)
    fire_gather(1, rows_b, gs_b)
    wait_gather(1, rows_b, gs_b)
    fire_scatter(rows_b, cidx_b, ss_b)
    wait_scatter(rows_a, cidx_a, ss_a)
    load_col(2, cidx_a)
    fire_gather(2, rows_a, gs_a)

    def body(i, carry):
        c = 2 * i + 2
        wait_gather(c, rows_a, gs_a)
        fire_scatter(rows_a, cidx_a, ss_a)
        wait_scatter(rows_b, cidx_b, ss_b)
        load_col(c + 1, cidx_b)
        fire_gather(c + 1, rows_b, gs_b)
        wait_gather(c + 1, rows_b, gs_b)
        fire_scatter(rows_b, cidx_b, ss_b)
        wait_scatter(rows_a, cidx_a, ss_a)

        @pl.when(c + 2 < NCH)
        def _():
            load_col(c + 2, cidx_a)
            fire_gather(c + 2, rows_a, gs_a)

        return carry

    lax.fori_loop(0, (NCH - 2) // 2, body, 0)
    wait_scatter(rows_b, cidx_b, ss_b)

    plsc.subcore_barrier()

    @pl.when(sid < NZ)
    def _():
        def wbody(j, carry):
            r0 = sid * ZB + j * WB
            pltpu.sync_copy(acc_sh.at[pl.ds(r0, WB)], rows_a.at[pl.ds(0, WB)])
            pltpu.sync_copy(rows_a.at[pl.ds(0, WB)],
                            accp_hbm.at[cid, pl.ds(r0, WB)])
            return carry

        lax.fori_loop(0, ZB // WB, wbody, 0)


_acc_call = pl.kernel(
    _acc_body,
    out_type=jax.ShapeDtypeStruct((NC, N_NODES, D), jnp.float32),
    mesh=_MESH,
    scratch_types=[
        pltpu.VMEM((NCH, K), jnp.int32),
        pltpu.VMEM((K,), jnp.int32),
        pltpu.VMEM((K,), jnp.int32),
        pltpu.VMEM((K, D), jnp.float32),
        pltpu.VMEM((K, D), jnp.float32),
        pltpu.VMEM_SHARED((NPAD, D), jnp.float32),
        pltpu.SemaphoreType.DMA,
        pltpu.SemaphoreType.DMA,
        pltpu.SemaphoreType.DMA,
        pltpu.SemaphoreType.DMA,
    ],
)


BLK = 1000


def _lin_body(x_ref, w_ref, bl_ref, degp_ref, g_ref):
    deg = degp_ref[:, 0] + degp_ref[:, 1]
    dis = jnp.where(deg > 0.0, lax.rsqrt(deg), 0.0)
    h = jnp.dot(x_ref[...], w_ref[...].T,
                preferred_element_type=jnp.float32) + bl_ref[...]
    g_ref[...] = h * dis[:, None]


_lin_call = pl.pallas_call(
    _lin_body,
    grid=(N_NODES // BLK,),
    in_specs=[
        pl.BlockSpec((BLK, D), lambda i: (i, 0)),
        pl.BlockSpec((D, D), lambda i: (0, 0)),
        pl.BlockSpec((1, D), lambda i: (0, 0)),
        pl.BlockSpec((BLK, NC), lambda i: (i, 0)),
    ],
    out_specs=pl.BlockSpec((BLK, D), lambda i: (i, 0)),
    out_shape=jax.ShapeDtypeStruct((N_NODES, D), jnp.float32),
)


def _out_body(accp_ref, degp_ref, bias_ref, out_ref):
    acc = accp_ref[0] + accp_ref[1]
    deg = degp_ref[:, 0] + degp_ref[:, 1]
    dis = jnp.where(deg > 0.0, lax.rsqrt(deg), 0.0)
    out_ref[...] = jnp.maximum(acc * dis[:, None] + bias_ref[...], 0.0)


_out_call = pl.pallas_call(
    _out_body,
    grid=(N_NODES // BLK,),
    in_specs=[
        pl.BlockSpec((NC, BLK, D), lambda i: (0, i, 0)),
        pl.BlockSpec((BLK, NC), lambda i: (i, 0)),
        pl.BlockSpec((1, D), lambda i: (0, 0)),
    ],
    out_specs=pl.BlockSpec((BLK, D), lambda i: (i, 0)),
    out_shape=jax.ShapeDtypeStruct((N_NODES, D), jnp.float32),
)


@jax.jit
def kernel(x, edge_index, W, b_lin, bias):
    rowp = jnp.pad(edge_index[0].reshape(NW, EPT), ((0, 0), (0, PAD)),
                   constant_values=0)
    colp = jnp.pad(edge_index[1].reshape(NW, EPT), ((0, 0), (0, PAD)),
                   constant_values=N_NODES)
    row3 = rowp.reshape(NW, NCH, K)
    col3 = colp.reshape(NW, NCH, K)
    colf = colp.reshape(NW * EPP)
    ones_k = jnp.ones((K,), jnp.float32)
    zeros_n = jnp.zeros((ZB,), jnp.float32)
    zrows = jnp.zeros((WB, D), jnp.float32)

    degp = _deg_call(col3, ones_k, zeros_n)
    degp_t = degp.reshape(NC, N_NODES).T
    g = _lin_call(x, W, b_lin.reshape(1, D), degp_t)
    accp = _acc_call(g, row3, colf, zrows)
    out = _out_call(accp, degp_t, bias.reshape(1, D))
    return out


# speedup vs baseline: 1.0062x; 1.0062x over previous
import functools

import jax
import jax.numpy as jnp
from jax import lax
from jax.experimental import pallas as pl
from jax.experimental.pallas import tpu as pltpu
from jax.experimental.pallas import tpu_sc as plsc

N_NODES = 10000
D = 128
E = 320000

NC = 2
NS = 16
NW = NC * NS
K = 128
EPT = E // NW
NCH = 80
EPP = NCH * K
PAD = EPP - EPT
NPAD = N_NODES + K
ZB = 1000
NZ = N_NODES // ZB
WB = 40
QD = 8

_MESH = plsc.VectorSubcoreMesh(core_axis_name="c", subcore_axis_name="s")


def _deg_body(col3_hbm, ones_hbm, zeros_hbm, degp_hbm, cidx_v, ones_v,
              stage_v, deg_sh, sem):
    cid = lax.axis_index("c")
    sid = lax.axis_index("s")
    wid = cid * NS + sid

    @pl.when(sid < NZ)
    def _():
        pltpu.sync_copy(zeros_hbm, stage_v)
        pltpu.sync_copy(stage_v, deg_sh.at[pl.ds(sid * ZB, ZB)])

    @pl.when(sid == NZ)
    def _():
        pltpu.sync_copy(zeros_hbm.at[pl.ds(0, NPAD - N_NODES)], stage_v.at[pl.ds(0, NPAD - N_NODES)])
        pltpu.sync_copy(stage_v.at[pl.ds(0, NPAD - N_NODES)],
                        deg_sh.at[pl.ds(N_NODES, NPAD - N_NODES)])

    pltpu.sync_copy(ones_hbm, ones_v)
    pltpu.sync_copy(col3_hbm.at[wid], cidx_v)
    plsc.subcore_barrier()

    def body(c, carry):
        pltpu.async_copy(ones_v, deg_sh.at[cidx_v.at[c]], sem, add=True)

        @pl.when(c >= QD)
        def _():
            pltpu.make_async_copy(ones_v, deg_sh.at[cidx_v.at[c - QD]],
                                  sem).wait()

        return carry

    lax.fori_loop(0, NCH, body, 0)

    def drain(c, carry):
        pltpu.make_async_copy(ones_v, deg_sh.at[cidx_v.at[c]], sem).wait()
        return carry

    lax.fori_loop(NCH - QD, NCH, drain, 0)

    plsc.subcore_barrier()

    @pl.when(sid < NZ)
    def _():
        pltpu.sync_copy(deg_sh.at[pl.ds(sid * ZB, ZB)], stage_v)
        pltpu.sync_copy(stage_v,
                        degp_hbm.at[pl.ds(cid * N_NODES + sid * ZB, ZB)])


_deg_call = pl.kernel(
    _deg_body,
    out_type=jax.ShapeDtypeStruct((NC * N_NODES,), jnp.float32),
    mesh=_MESH,
    scratch_types=[
        pltpu.VMEM((NCH, K), jnp.int32),
        pltpu.VMEM((K,), jnp.float32),
        pltpu.VMEM((ZB,), jnp.float32),
        pltpu.VMEM_SHARED((NPAD,), jnp.float32),
        pltpu.SemaphoreType.DMA,
    ],
)


def _acc_body(g_hbm, row3_hbm, colf_hbm, zrows_hbm, accp_hbm, ridx_v,
              cidx_a, cidx_b, rows_a, rows_b, acc_sh,
              gs_a, gs_b, ss_a, ss_b):
    cid = lax.axis_index("c")
    sid = lax.axis_index("s")
    wid = cid * NS + sid
    ebase = wid * EPP

    pltpu.sync_copy(zrows_hbm, rows_a)

    @pl.when(sid < NZ)
    def _():
        def zbody(j, carry):
            pltpu.sync_copy(rows_a.at[pl.ds(0, WB)],
                            acc_sh.at[pl.ds(sid * ZB + j * WB, WB)])
            return carry

        lax.fori_loop(0, ZB // WB, zbody, 0)

    @pl.when(sid == NZ)
    def _():
        pltpu.sync_copy(rows_a, acc_sh.at[pl.ds(N_NODES, NPAD - N_NODES)])

    pltpu.sync_copy(row3_hbm.at[wid], ridx_v)
    plsc.subcore_barrier()

    def fire_gather(c, rows, sem):
        pltpu.async_copy(g_hbm.at[ridx_v.at[c]], rows, sem)

    def wait_gather(c, rows, sem):
        pltpu.make_async_copy(g_hbm.at[ridx_v.at[c]], rows, sem).wait()

    def load_col(c, cidx):
        pltpu.sync_copy(colf_hbm.at[pl.ds(ebase + c * K, K)], cidx)

    def fire_scatter(rows, cidx, sem):
        pltpu.async_copy(rows, acc_sh.at[cidx], sem, add=True)

    def wait_scatter(rows, cidx, sem):
        pltpu.make_async_copy(rows, acc_sh.at[cidx], sem).wait()

    load_col(0, cidx_a)
    fire_gather(0, rows_a, gs_a)
    load_col(1, cidx_b)
    wait_gather(0, rows_a, gs_a)
    fire_scatter(rows_a, cidx_a, ss_a)
    fire_gather(1, rows_b, gs_b)
    wait_gather(1, rows_b, gs_b)
    fire_scatter(rows_b, cidx_b, ss_b)
    wait_scatter(rows_a, cidx_a, ss_a)
    load_col(2, cidx_a)
    fire_gather(2, rows_a, gs_a)

    def body(i, carry):
        c = 2 * i + 2
        wait_gather(c, rows_a, gs_a)
        fire_scatter(rows_a, cidx_a, ss_a)
        wait_scatter(rows_b, cidx_b, ss_b)
        load_col(c + 1, cidx_b)
        fire_gather(c + 1, rows_b, gs_b)
        wait_gather(c + 1, rows_b, gs_b)
        fire_scatter(rows_b, cidx_b, ss_b)
        wait_scatter(rows_a, cidx_a, ss_a)

        @pl.when(c + 2 < NCH)
        def _():
            load_col(c + 2, cidx_a)
            fire_gather(c + 2, rows_a, gs_a)

        return carry

    lax.fori_loop(0, (NCH - 2) // 2, body, 0)
    wait_scatter(rows_b, cidx_b, ss_b)

    plsc.subcore_barrier()

    @pl.when(sid < NZ)
    def _():
        def wbody(j, carry):
            r0 = sid * ZB + j * WB
            pltpu.sync_copy(acc_sh.at[pl.ds(r0, WB)], rows_a.at[pl.ds(0, WB)])
            pltpu.sync_copy(rows_a.at[pl.ds(0, WB)],
                            accp_hbm.at[cid, pl.ds(r0, WB)])
            return carry

        lax.fori_loop(0, ZB // WB, wbody, 0)


_acc_call = pl.kernel(
    _acc_body,
    out_type=jax.ShapeDtypeStruct((NC, N_NODES, D), jnp.float32),
    mesh=_MESH,
    scratch_types=[
        pltpu.VMEM((NCH, K), jnp.int32),
        pltpu.VMEM((K,), jnp.int32),
        pltpu.VMEM((K,), jnp.int32),
        pltpu.VMEM((K, D), jnp.float32),
        pltpu.VMEM((K, D), jnp.float32),
        pltpu.VMEM_SHARED((NPAD, D), jnp.float32),
        pltpu.SemaphoreType.DMA,
        pltpu.SemaphoreType.DMA,
        pltpu.SemaphoreType.DMA,
        pltpu.SemaphoreType.DMA,
    ],
)


BLK = 1000


def _lin_body(x_ref, w_ref, bl_ref, degp_ref, g_ref):
    deg = degp_ref[:, 0] + degp_ref[:, 1]
    dis = jnp.where(deg > 0.0, lax.rsqrt(deg), 0.0)
    h = jnp.dot(x_ref[...], w_ref[...].T,
                preferred_element_type=jnp.float32) + bl_ref[...]
    g_ref[...] = h * dis[:, None]


_lin_call = pl.pallas_call(
    _lin_body,
    grid=(N_NODES // BLK,),
    in_specs=[
        pl.BlockSpec((BLK, D), lambda i: (i, 0)),
        pl.BlockSpec((D, D), lambda i: (0, 0)),
        pl.BlockSpec((1, D), lambda i: (0, 0)),
        pl.BlockSpec((BLK, NC), lambda i: (i, 0)),
    ],
    out_specs=pl.BlockSpec((BLK, D), lambda i: (i, 0)),
    out_shape=jax.ShapeDtypeStruct((N_NODES, D), jnp.float32),
)


def _out_body(accp_ref, degp_ref, bias_ref, out_ref):
    acc = accp_ref[0] + accp_ref[1]
    deg = degp_ref[:, 0] + degp_ref[:, 1]
    dis = jnp.where(deg > 0.0, lax.rsqrt(deg), 0.0)
    out_ref[...] = jnp.maximum(acc * dis[:, None] + bias_ref[...], 0.0)


_out_call = pl.pallas_call(
    _out_body,
    grid=(N_NODES // BLK,),
    in_specs=[
        pl.BlockSpec((NC, BLK, D), lambda i: (0, i, 0)),
        pl.BlockSpec((BLK, NC), lambda i: (i, 0)),
        pl.BlockSpec((1, D), lambda i: (0, 0)),
    ],
    out_specs=pl.BlockSpec((BLK, D), lambda i: (i, 0)),
    out_shape=jax.ShapeDtypeStruct((N_NODES, D), jnp.float32),
)


@jax.jit
def kernel(x, edge_index, W, b_lin, bias):
    rowp = jnp.pad(edge_index[0].reshape(NW, EPT), ((0, 0), (0, PAD)),
                   constant_values=0)
    padcol = N_NODES + (jnp.arange(PAD, dtype=jnp.int32) % K)
    colp = jnp.concatenate(
        [edge_index[1].reshape(NW, EPT),
         jnp.broadcast_to(padcol, (NW, PAD))], axis=1)
    row3 = rowp.reshape(NW, NCH, K)
    col3 = colp.reshape(NW, NCH, K)
    colf = colp.reshape(NW * EPP)
    ones_k = jnp.ones((K,), jnp.float32)
    zeros_n = jnp.zeros((ZB,), jnp.float32)
    zrows = jnp.zeros((K, D), jnp.float32)

    degp = _deg_call(col3, ones_k, zeros_n)
    degp_t = degp.reshape(NC, N_NODES).T
    g = _lin_call(x, W, b_lin.reshape(1, D), degp_t)
    accp = _acc_call(g, row3, colf, zrows)
    out = _out_call(accp, degp_t, bias.reshape(1, D))
    return out


# speedup vs baseline: 1.0667x; 1.0601x over previous
import functools

import jax
import jax.numpy as jnp
from jax import lax
from jax.experimental import pallas as pl
from jax.experimental.pallas import tpu as pltpu
from jax.experimental.pallas import tpu_sc as plsc

N_NODES = 10000
D = 128
E = 320000

NC = 2
NS = 16
NW = NC * NS
K = 128
EPT = E // NW
NCH = 80
EPP = NCH * K
PAD = EPP - EPT
NPAD = N_NODES + K
ZB = 1000
NZ = N_NODES // ZB
WB = 40
QD = 8

_MESH = plsc.VectorSubcoreMesh(core_axis_name="c", subcore_axis_name="s")


def _deg_body(col3_hbm, ones_hbm, zeros_hbm, degp_hbm, cidx_v, ones_v,
              stage_v, deg_sh, sem):
    cid = lax.axis_index("c")
    sid = lax.axis_index("s")
    wid = cid * NS + sid

    @pl.when(sid < NZ)
    def _():
        pltpu.sync_copy(zeros_hbm, stage_v)
        pltpu.sync_copy(stage_v, deg_sh.at[pl.ds(sid * ZB, ZB)])

    @pl.when(sid == NZ)
    def _():
        pltpu.sync_copy(zeros_hbm.at[pl.ds(0, NPAD - N_NODES)], stage_v.at[pl.ds(0, NPAD - N_NODES)])
        pltpu.sync_copy(stage_v.at[pl.ds(0, NPAD - N_NODES)],
                        deg_sh.at[pl.ds(N_NODES, NPAD - N_NODES)])

    pltpu.sync_copy(ones_hbm, ones_v)
    pltpu.sync_copy(col3_hbm.at[wid], cidx_v)
    plsc.subcore_barrier()

    def body(c, carry):
        pltpu.async_copy(ones_v, deg_sh.at[cidx_v.at[c]], sem, add=True)

        @pl.when(c >= QD)
        def _():
            pltpu.make_async_copy(ones_v, deg_sh.at[cidx_v.at[c - QD]],
                                  sem).wait()

        return carry

    lax.fori_loop(0, NCH, body, 0)

    def drain(c, carry):
        pltpu.make_async_copy(ones_v, deg_sh.at[cidx_v.at[c]], sem).wait()
        return carry

    lax.fori_loop(NCH - QD, NCH, drain, 0)

    plsc.subcore_barrier()

    @pl.when(sid < NZ)
    def _():
        pltpu.sync_copy(deg_sh.at[pl.ds(sid * ZB, ZB)], stage_v)
        pltpu.sync_copy(stage_v,
                        degp_hbm.at[pl.ds(cid * N_NODES + sid * ZB, ZB)])


_deg_call = pl.kernel(
    _deg_body,
    out_type=jax.ShapeDtypeStruct((NC * N_NODES,), jnp.float32),
    mesh=_MESH,
    scratch_types=[
        pltpu.VMEM((NCH, K), jnp.int32),
        pltpu.VMEM((K,), jnp.float32),
        pltpu.VMEM((ZB,), jnp.float32),
        pltpu.VMEM_SHARED((NPAD,), jnp.float32),
        pltpu.SemaphoreType.DMA,
    ],
)


def _acc_body(g_hbm, rowf_hbm, colf_hbm, zrows_hbm, accp_hbm, ridx_a,
              ridx_b, cidx_a, cidx_b, rows_a, rows_b, acc_sh,
              gs_a, gs_b, is_ra, is_ca, is_rb, is_cb):
    cid = lax.axis_index("c")
    sid = lax.axis_index("s")
    wid = cid * NS + sid
    ebase = wid * EPP

    pltpu.sync_copy(zrows_hbm, rows_a)

    @pl.when(sid < NZ)
    def _():
        def zbody(j, carry):
            pltpu.sync_copy(rows_a.at[pl.ds(0, WB)],
                            acc_sh.at[pl.ds(sid * ZB + j * WB, WB)])
            return carry

        lax.fori_loop(0, ZB // WB, zbody, 0)

    @pl.when(sid == NZ)
    def _():
        pltpu.sync_copy(rows_a, acc_sh.at[pl.ds(N_NODES, NPAD - N_NODES)])

    plsc.subcore_barrier()

    def fire_idx(c, ridx, cidx, sem_r, sem_c):
        pltpu.async_copy(rowf_hbm.at[pl.ds(ebase + c * K, K)], ridx, sem_r)
        pltpu.async_copy(colf_hbm.at[pl.ds(ebase + c * K, K)], cidx, sem_c)

    def wait_idx(c, ridx, cidx, sem_r, sem_c):
        pltpu.make_async_copy(rowf_hbm.at[pl.ds(ebase + c * K, K)], ridx,
                              sem_r).wait()
        pltpu.make_async_copy(colf_hbm.at[pl.ds(ebase + c * K, K)], cidx,
                              sem_c).wait()

    def fire_gather(rows, ridx, sem):
        pltpu.async_copy(g_hbm.at[ridx], rows, sem)

    def wait_gather(rows, ridx, sem):
        pltpu.make_async_copy(g_hbm.at[ridx], rows, sem).wait()

    def scatter(rows, cidx):
        pltpu.sync_copy(rows, acc_sh.at[cidx], add=True)

    fire_idx(0, ridx_a, cidx_a, is_ra, is_ca)
    wait_idx(0, ridx_a, cidx_a, is_ra, is_ca)
    fire_gather(rows_a, ridx_a, gs_a)
    fire_idx(1, ridx_b, cidx_b, is_rb, is_cb)

    def body(i, carry):
        c = 2 * i
        wait_gather(rows_a, ridx_a, gs_a)
        wait_idx(c + 1, ridx_b, cidx_b, is_rb, is_cb)
        fire_gather(rows_b, ridx_b, gs_b)
        scatter(rows_a, cidx_a)

        @pl.when(c + 2 < NCH)
        def _():
            fire_idx(c + 2, ridx_a, cidx_a, is_ra, is_ca)

        wait_gather(rows_b, ridx_b, gs_b)

        @pl.when(c + 2 < NCH)
        def _():
            wait_idx(c + 2, ridx_a, cidx_a, is_ra, is_ca)
            fire_gather(rows_a, ridx_a, gs_a)

        scatter(rows_b, cidx_b)

        @pl.when(c + 3 < NCH)
        def _():
            fire_idx(c + 3, ridx_b, cidx_b, is_rb, is_cb)

        return carry

    lax.fori_loop(0, NCH // 2, body, 0)

    plsc.subcore_barrier()

    @pl.when(sid < NZ)
    def _():
        def wbody(j, carry):
            r0 = sid * ZB + j * WB
            pltpu.sync_copy(acc_sh.at[pl.ds(r0, WB)], rows_a.at[pl.ds(0, WB)])
            pltpu.sync_copy(rows_a.at[pl.ds(0, WB)],
                            accp_hbm.at[cid, pl.ds(r0, WB)])
            return carry

        lax.fori_loop(0, ZB // WB, wbody, 0)


_acc_call = pl.kernel(
    _acc_body,
    out_type=jax.ShapeDtypeStruct((NC, N_NODES, D), jnp.float32),
    mesh=_MESH,
    scratch_types=[
        pltpu.VMEM((K,), jnp.int32),
        pltpu.VMEM((K,), jnp.int32),
        pltpu.VMEM((K,), jnp.int32),
        pltpu.VMEM((K,), jnp.int32),
        pltpu.VMEM((K, D), jnp.float32),
        pltpu.VMEM((K, D), jnp.float32),
        pltpu.VMEM_SHARED((NPAD, D), jnp.float32),
        pltpu.SemaphoreType.DMA,
        pltpu.SemaphoreType.DMA,
        pltpu.SemaphoreType.DMA,
        pltpu.SemaphoreType.DMA,
        pltpu.SemaphoreType.DMA,
        pltpu.SemaphoreType.DMA,
    ],
)


BLK = 1000


def _lin_body(x_ref, w_ref, bl_ref, degp_ref, g_ref):
    deg = degp_ref[:, 0] + degp_ref[:, 1]
    dis = jnp.where(deg > 0.0, lax.rsqrt(deg), 0.0)
    h = jnp.dot(x_ref[...], w_ref[...].T,
                preferred_element_type=jnp.float32) + bl_ref[...]
    g_ref[...] = h * dis[:, None]


_lin_call = pl.pallas_call(
    _lin_body,
    grid=(N_NODES // BLK,),
    in_specs=[
        pl.BlockSpec((BLK, D), lambda i: (i, 0)),
        pl.BlockSpec((D, D), lambda i: (0, 0)),
        pl.BlockSpec((1, D), lambda i: (0, 0)),
        pl.BlockSpec((BLK, NC), lambda i: (i, 0)),
    ],
    out_specs=pl.BlockSpec((BLK, D), lambda i: (i, 0)),
    out_shape=jax.ShapeDtypeStruct((N_NODES, D), jnp.float32),
)


def _out_body(accp_ref, degp_ref, bias_ref, out_ref):
    acc = accp_ref[0] + accp_ref[1]
    deg = degp_ref[:, 0] + degp_ref[:, 1]
    dis = jnp.where(deg > 0.0, lax.rsqrt(deg), 0.0)
    out_ref[...] = jnp.maximum(acc * dis[:, None] + bias_ref[...], 0.0)


_out_call = pl.pallas_call(
    _out_body,
    grid=(N_NODES // BLK,),
    in_specs=[
        pl.BlockSpec((NC, BLK, D), lambda i: (0, i, 0)),
        pl.BlockSpec((BLK, NC), lambda i: (i, 0)),
        pl.BlockSpec((1, D), lambda i: (0, 0)),
    ],
    out_specs=pl.BlockSpec((BLK, D), lambda i: (i, 0)),
    out_shape=jax.ShapeDtypeStruct((N_NODES, D), jnp.float32),
)


@jax.jit
def kernel(x, edge_index, W, b_lin, bias):
    rowp = jnp.pad(edge_index[0].reshape(NW, EPT), ((0, 0), (0, PAD)),
                   constant_values=0)
    padcol = N_NODES + (jnp.arange(PAD, dtype=jnp.int32) % K)
    colp = jnp.concatenate(
        [edge_index[1].reshape(NW, EPT),
         jnp.broadcast_to(padcol, (NW, PAD))], axis=1)
    col3 = colp.reshape(NW, NCH, K)
    rowf = rowp.reshape(NW * EPP)
    colf = colp.reshape(NW * EPP)
    ones_k = jnp.ones((K,), jnp.float32)
    zeros_n = jnp.zeros((ZB,), jnp.float32)
    zrows = jnp.zeros((K, D), jnp.float32)

    degp = _deg_call(col3, ones_k, zeros_n)
    degp_t = degp.reshape(NC, N_NODES).T
    g = _lin_call(x, W, b_lin.reshape(1, D), degp_t)
    accp = _acc_call(g, rowf, colf, zrows)
    out = _out_call(accp, degp_t, bias.reshape(1, D))
    return out


# speedup vs baseline: 2.3448x; 2.1982x over previous
import functools

import jax
import jax.numpy as jnp
from jax import lax
from jax.experimental import pallas as pl
from jax.experimental.pallas import tpu as pltpu
from jax.experimental.pallas import tpu_sc as plsc

N_NODES = 10000
D = 128
E = 320000

NC = 2
NS = 16
NW = NC * NS
EPT = E // NW
K = 128
FULL = EPT // K
TAIL = EPT - FULL * K
ZB = 1000
NZ = N_NODES // ZB

_MESH = plsc.VectorSubcoreMesh(core_axis_name="c", subcore_axis_name="s")


NCH = 80
EPP = NCH * K
PAD = EPP - EPT
NPAD = N_NODES + K
QD = 8


def _deg_body(col3_hbm, ones_hbm, zeros_hbm, degp_hbm, cidx_v, ones_v,
              stage_v, deg_sh, sem):
    cid = lax.axis_index("c")
    sid = lax.axis_index("s")
    wid = cid * NS + sid

    @pl.when(sid < NZ)
    def _():
        pltpu.sync_copy(zeros_hbm, stage_v)
        pltpu.sync_copy(stage_v, deg_sh.at[pl.ds(sid * ZB, ZB)])

    @pl.when(sid == NZ)
    def _():
        pltpu.sync_copy(zeros_hbm.at[pl.ds(0, K)], stage_v.at[pl.ds(0, K)])
        pltpu.sync_copy(stage_v.at[pl.ds(0, K)],
                        deg_sh.at[pl.ds(N_NODES, K)])

    pltpu.sync_copy(ones_hbm, ones_v)
    pltpu.sync_copy(col3_hbm.at[wid], cidx_v)
    plsc.subcore_barrier()

    def body(c, carry):
        pltpu.async_copy(ones_v, deg_sh.at[cidx_v.at[c]], sem, add=True)

        @pl.when(c >= QD)
        def _():
            pltpu.make_async_copy(ones_v, deg_sh.at[cidx_v.at[c - QD]],
                                  sem).wait()

        return carry

    lax.fori_loop(0, NCH, body, 0)

    def drain(c, carry):
        pltpu.make_async_copy(ones_v, deg_sh.at[cidx_v.at[c]], sem).wait()
        return carry

    lax.fori_loop(NCH - QD, NCH, drain, 0)

    plsc.subcore_barrier()

    @pl.when(sid < NZ)
    def _():
        pltpu.sync_copy(deg_sh.at[pl.ds(sid * ZB, ZB)], stage_v)
        pltpu.sync_copy(stage_v,
                        degp_hbm.at[pl.ds(cid * N_NODES + sid * ZB, ZB)])


_deg_call = pl.kernel(
    _deg_body,
    out_type=jax.ShapeDtypeStruct((NC * N_NODES,), jnp.float32),
    mesh=_MESH,
    scratch_types=[
        pltpu.VMEM((NCH, K), jnp.int32),
        pltpu.VMEM((K,), jnp.float32),
        pltpu.VMEM((ZB,), jnp.float32),
        pltpu.VMEM_SHARED((NPAD,), jnp.float32),
        pltpu.SemaphoreType.DMA,
    ],
)


WB = 40


def _acc_body(g_hbm, row_hbm, col_hbm, zrows_hbm, accp_hbm, ridx_a, cidx_a,
              ridx_b, cidx_b, ridx_t, cidx_t, rows_a, rows_b, rows_t, zb_v,
              acc_sh, sem_a, sem_b):
    cid = lax.axis_index("c")
    sid = lax.axis_index("s")
    base = (cid * NS + sid) * EPT

    @pl.when(sid < NZ)
    def _():
        pltpu.sync_copy(zrows_hbm, zb_v)

        def zbody(j, carry):
            pltpu.sync_copy(zb_v, acc_sh.at[pl.ds(sid * ZB + j * WB, WB)])
            return carry

        lax.fori_loop(0, ZB // WB, zbody, 0)

    plsc.subcore_barrier()

    def load_idx(c, ridx, cidx):
        e0 = base + c * K
        pltpu.sync_copy(row_hbm.at[pl.ds(e0, K)], ridx)
        pltpu.sync_copy(col_hbm.at[pl.ds(e0, K)], cidx)

    load_idx(0, ridx_a, cidx_a)
    pltpu.async_copy(g_hbm.at[ridx_a], rows_a, sem_a)
    load_idx(1, ridx_b, cidx_b)

    def body(i, carry):
        a = 2 * i
        pltpu.make_async_copy(g_hbm.at[ridx_a], rows_a, sem_a).wait()
        pltpu.async_copy(g_hbm.at[ridx_b], rows_b, sem_b)
        pltpu.sync_copy(rows_a, acc_sh.at[cidx_a], add=True)
        load_idx(a + 2, ridx_a, cidx_a)
        pltpu.make_async_copy(g_hbm.at[ridx_b], rows_b, sem_b).wait()
        pltpu.async_copy(g_hbm.at[ridx_a], rows_a, sem_a)
        pltpu.sync_copy(rows_b, acc_sh.at[cidx_b], add=True)
        load_idx(a + 3, ridx_b, cidx_b)
        return carry

    lax.fori_loop(0, FULL // 2 - 1, body, 0)
    pltpu.make_async_copy(g_hbm.at[ridx_a], rows_a, sem_a).wait()
    pltpu.async_copy(g_hbm.at[ridx_b], rows_b, sem_b)
    pltpu.sync_copy(rows_a, acc_sh.at[cidx_a], add=True)
    e0 = base + FULL * K
    pltpu.sync_copy(row_hbm.at[pl.ds(e0, TAIL)], ridx_t)
    pltpu.sync_copy(col_hbm.at[pl.ds(e0, TAIL)], cidx_t)
    pltpu.make_async_copy(g_hbm.at[ridx_b], rows_b, sem_b).wait()
    pltpu.async_copy(g_hbm.at[ridx_t], rows_t, sem_a)
    pltpu.sync_copy(rows_b, acc_sh.at[cidx_b], add=True)
    pltpu.make_async_copy(g_hbm.at[ridx_t], rows_t, sem_a).wait()
    pltpu.sync_copy(rows_t, acc_sh.at[cidx_t], add=True)

    plsc.subcore_barrier()

    @pl.when(sid < NZ)
    def _():
        def wbody(j, carry):
            r0 = sid * ZB + j * WB
            pltpu.sync_copy(acc_sh.at[pl.ds(r0, WB)], zb_v)
            pltpu.sync_copy(zb_v, accp_hbm.at[cid, pl.ds(r0, WB)])
            return carry

        lax.fori_loop(0, ZB // WB, wbody, 0)


_acc_call = pl.kernel(
    _acc_body,
    out_type=jax.ShapeDtypeStruct((NC, N_NODES, D), jnp.float32),
    mesh=_MESH,
    scratch_types=[
        pltpu.VMEM((K,), jnp.int32),
        pltpu.VMEM((K,), jnp.int32),
        pltpu.VMEM((K,), jnp.int32),
        pltpu.VMEM((K,), jnp.int32),
        pltpu.VMEM((TAIL,), jnp.int32),
        pltpu.VMEM((TAIL,), jnp.int32),
        pltpu.VMEM((K, D), jnp.float32),
        pltpu.VMEM((K, D), jnp.float32),
        pltpu.VMEM((TAIL, D), jnp.float32),
        pltpu.VMEM((WB, D), jnp.float32),
        pltpu.VMEM_SHARED((N_NODES, D), jnp.float32),
        pltpu.SemaphoreType.DMA,
        pltpu.SemaphoreType.DMA,
    ],
)


BLK = 1000


def _lin_body(x_ref, w_ref, bl_ref, degp_ref, g_ref):
    deg = degp_ref[:, 0] + degp_ref[:, 1]
    dis = jnp.where(deg > 0.0, lax.rsqrt(deg), 0.0)
    h = jnp.dot(x_ref[...], w_ref[...].T,
                preferred_element_type=jnp.float32) + bl_ref[...]
    g_ref[...] = h * dis[:, None]


_lin_call = pl.pallas_call(
    _lin_body,
    grid=(N_NODES // BLK,),
    in_specs=[
        pl.BlockSpec((BLK, D), lambda i: (i, 0)),
        pl.BlockSpec((D, D), lambda i: (0, 0)),
        pl.BlockSpec((1, D), lambda i: (0, 0)),
        pl.BlockSpec((BLK, NC), lambda i: (i, 0)),
    ],
    out_specs=pl.BlockSpec((BLK, D), lambda i: (i, 0)),
    out_shape=jax.ShapeDtypeStruct((N_NODES, D), jnp.float32),
)


def _out_body(accp_ref, degp_ref, bias_ref, out_ref):
    acc = accp_ref[0] + accp_ref[1]
    deg = degp_ref[:, 0] + degp_ref[:, 1]
    dis = jnp.where(deg > 0.0, lax.rsqrt(deg), 0.0)
    out_ref[...] = jnp.maximum(acc * dis[:, None] + bias_ref[...], 0.0)


_out_call = pl.pallas_call(
    _out_body,
    grid=(N_NODES // BLK,),
    in_specs=[
        pl.BlockSpec((NC, BLK, D), lambda i: (0, i, 0)),
        pl.BlockSpec((BLK, NC), lambda i: (i, 0)),
        pl.BlockSpec((1, D), lambda i: (0, 0)),
    ],
    out_specs=pl.BlockSpec((BLK, D), lambda i: (i, 0)),
    out_shape=jax.ShapeDtypeStruct((N_NODES, D), jnp.float32),
)


@jax.jit
def kernel(x, edge_index, W, b_lin, bias):
    row = edge_index[0]
    col = edge_index[1]
    ones_k = jnp.ones((K,), jnp.float32)
    zeros_n = jnp.zeros((ZB,), jnp.float32)
    zrows = jnp.zeros((WB, D), jnp.float32)

    padcol = N_NODES + (jnp.arange(PAD, dtype=jnp.int32) % K)
    colp = jnp.concatenate(
        [col.reshape(NW, EPT), jnp.broadcast_to(padcol, (NW, PAD))], axis=1)
    col3 = colp.reshape(NW, NCH, K)

    degp = _deg_call(col3, ones_k, zeros_n)
    degp_t = degp.reshape(NC, N_NODES).T
    g = _lin_call(x, W, b_lin.reshape(1, D), degp_t)
    accp = _acc_call(g, row, col, zrows)
    out = _out_call(accp, degp_t, bias.reshape(1, D))
    return out


# speedup vs baseline: 2.5712x; 1.0966x over previous
import functools

import jax
import jax.numpy as jnp
from jax import lax
from jax.experimental import pallas as pl
from jax.experimental.pallas import tpu as pltpu
from jax.experimental.pallas import tpu_sc as plsc

N_NODES = 10000
D = 128
E = 320000

NC = 2
NS = 16
NW = NC * NS
EPT = E // NW
K = 128
FULL = EPT // K
TAIL = EPT - FULL * K
ZB = 1000
NZ = N_NODES // ZB

_MESH = plsc.VectorSubcoreMesh(core_axis_name="c", subcore_axis_name="s")


NCH = 80
EPP = NCH * K
PAD = EPP - EPT
NPAD = N_NODES + K
QD = 8


def _deg_body(col3_hbm, ones_hbm, zeros_hbm, degp_hbm, cidx_v, ones_v,
              stage_v, deg_sh, sem):
    cid = lax.axis_index("c")
    sid = lax.axis_index("s")
    wid = cid * NS + sid

    @pl.when(sid < NZ)
    def _():
        pltpu.sync_copy(zeros_hbm, stage_v)
        pltpu.sync_copy(stage_v, deg_sh.at[pl.ds(sid * ZB, ZB)])

    @pl.when(sid == NZ)
    def _():
        pltpu.sync_copy(zeros_hbm.at[pl.ds(0, K)], stage_v.at[pl.ds(0, K)])
        pltpu.sync_copy(stage_v.at[pl.ds(0, K)],
                        deg_sh.at[pl.ds(N_NODES, K)])

    pltpu.sync_copy(ones_hbm, ones_v)
    pltpu.sync_copy(col3_hbm.at[wid], cidx_v)
    plsc.subcore_barrier()

    def body(c, carry):
        pltpu.async_copy(ones_v, deg_sh.at[cidx_v.at[c]], sem, add=True)

        @pl.when(c >= QD)
        def _():
            pltpu.make_async_copy(ones_v, deg_sh.at[cidx_v.at[c - QD]],
                                  sem).wait()

        return carry

    lax.fori_loop(0, NCH, body, 0)

    def drain(c, carry):
        pltpu.make_async_copy(ones_v, deg_sh.at[cidx_v.at[c]], sem).wait()
        return carry

    lax.fori_loop(NCH - QD, NCH, drain, 0)

    plsc.subcore_barrier()

    @pl.when(sid < NZ)
    def _():
        pltpu.sync_copy(deg_sh.at[pl.ds(sid * ZB, ZB)], stage_v)
        pltpu.sync_copy(stage_v,
                        degp_hbm.at[pl.ds(cid * N_NODES + sid * ZB, ZB)])


_deg_call = pl.kernel(
    _deg_body,
    out_type=jax.ShapeDtypeStruct((NC * N_NODES,), jnp.float32),
    mesh=_MESH,
    scratch_types=[
        pltpu.VMEM((NCH, K), jnp.int32),
        pltpu.VMEM((K,), jnp.float32),
        pltpu.VMEM((ZB,), jnp.float32),
        pltpu.VMEM_SHARED((NPAD,), jnp.float32),
        pltpu.SemaphoreType.DMA,
    ],
)


WB = 40


def _acc_body(g_hbm, row_hbm, col_hbm, zrows_hbm, accp_hbm, ridx_a, cidx_a,
              ridx_b, cidx_b, ridx_t, cidx_t, rows_a, rows_b, rows_t, zb_v,
              acc_sh, sem_a, sem_b, sem_ia, sem_ib):
    cid = lax.axis_index("c")
    sid = lax.axis_index("s")
    base = (cid * NS + sid) * EPT

    @pl.when(sid < NZ)
    def _():
        pltpu.sync_copy(zrows_hbm, zb_v)

        def zbody(j, carry):
            pltpu.sync_copy(zb_v, acc_sh.at[pl.ds(sid * ZB + j * WB, WB)])
            return carry

        lax.fori_loop(0, ZB // WB, zbody, 0)

    plsc.subcore_barrier()

    def load_idx(c, ridx, cidx):
        e0 = base + c * K
        pltpu.sync_copy(row_hbm.at[pl.ds(e0, K)], ridx)
        pltpu.sync_copy(col_hbm.at[pl.ds(e0, K)], cidx)

    def fire_idx(c, ridx, cidx, sem):
        e0 = base + c * K
        pltpu.async_copy(row_hbm.at[pl.ds(e0, K)], ridx, sem)
        pltpu.async_copy(col_hbm.at[pl.ds(e0, K)], cidx, sem)

    def wait_idx(c, ridx, cidx, sem):
        e0 = base + c * K
        pltpu.make_async_copy(row_hbm.at[pl.ds(e0, K)], ridx, sem).wait()
        pltpu.make_async_copy(col_hbm.at[pl.ds(e0, K)], cidx, sem).wait()

    load_idx(0, ridx_a, cidx_a)
    pltpu.async_copy(g_hbm.at[ridx_a], rows_a, sem_a)
    fire_idx(1, ridx_b, cidx_b, sem_ib)

    def body(i, carry):
        a = 2 * i
        pltpu.make_async_copy(g_hbm.at[ridx_a], rows_a, sem_a).wait()
        wait_idx(a + 1, ridx_b, cidx_b, sem_ib)
        pltpu.async_copy(g_hbm.at[ridx_b], rows_b, sem_b)
        pltpu.sync_copy(rows_a, acc_sh.at[cidx_a], add=True)
        fire_idx(a + 2, ridx_a, cidx_a, sem_ia)
        pltpu.make_async_copy(g_hbm.at[ridx_b], rows_b, sem_b).wait()
        wait_idx(a + 2, ridx_a, cidx_a, sem_ia)
        pltpu.async_copy(g_hbm.at[ridx_a], rows_a, sem_a)
        pltpu.sync_copy(rows_b, acc_sh.at[cidx_b], add=True)
        fire_idx(a + 3, ridx_b, cidx_b, sem_ib)
        return carry

    lax.fori_loop(0, FULL // 2 - 1, body, 0)
    wait_idx(FULL - 1, ridx_b, cidx_b, sem_ib)
    pltpu.make_async_copy(g_hbm.at[ridx_a], rows_a, sem_a).wait()
    pltpu.async_copy(g_hbm.at[ridx_b], rows_b, sem_b)
    pltpu.sync_copy(rows_a, acc_sh.at[cidx_a], add=True)
    e0 = base + FULL * K
    pltpu.sync_copy(row_hbm.at[pl.ds(e0, TAIL)], ridx_t)
    pltpu.sync_copy(col_hbm.at[pl.ds(e0, TAIL)], cidx_t)
    pltpu.make_async_copy(g_hbm.at[ridx_b], rows_b, sem_b).wait()
    pltpu.async_copy(g_hbm.at[ridx_t], rows_t, sem_a)
    pltpu.sync_copy(rows_b, acc_sh.at[cidx_b], add=True)
    pltpu.make_async_copy(g_hbm.at[ridx_t], rows_t, sem_a).wait()
    pltpu.sync_copy(rows_t, acc_sh.at[cidx_t], add=True)

    plsc.subcore_barrier()

    @pl.when(sid < NZ)
    def _():
        def wbody(j, carry):
            r0 = sid * ZB + j * WB
            pltpu.sync_copy(acc_sh.at[pl.ds(r0, WB)], zb_v)
            pltpu.sync_copy(zb_v, accp_hbm.at[cid, pl.ds(r0, WB)])
            return carry

        lax.fori_loop(0, ZB // WB, wbody, 0)


_acc_call = pl.kernel(
    _acc_body,
    out_type=jax.ShapeDtypeStruct((NC, N_NODES, D), jnp.float32),
    mesh=_MESH,
    scratch_types=[
        pltpu.VMEM((K,), jnp.int32),
        pltpu.VMEM((K,), jnp.int32),
        pltpu.VMEM((K,), jnp.int32),
        pltpu.VMEM((K,), jnp.int32),
        pltpu.VMEM((TAIL,), jnp.int32),
        pltpu.VMEM((TAIL,), jnp.int32),
        pltpu.VMEM((K, D), jnp.float32),
        pltpu.VMEM((K, D), jnp.float32),
        pltpu.VMEM((TAIL, D), jnp.float32),
        pltpu.VMEM((WB, D), jnp.float32),
        pltpu.VMEM_SHARED((N_NODES, D), jnp.float32),
        pltpu.SemaphoreType.DMA,
        pltpu.SemaphoreType.DMA,
        pltpu.SemaphoreType.DMA,
        pltpu.SemaphoreType.DMA,
    ],
)


BLK = 1000


def _lin_body(x_ref, w_ref, bl_ref, degp_ref, g_ref):
    deg = degp_ref[:, 0] + degp_ref[:, 1]
    dis = jnp.where(deg > 0.0, lax.rsqrt(deg), 0.0)
    h = jnp.dot(x_ref[...], w_ref[...].T,
                preferred_element_type=jnp.float32) + bl_ref[...]
    g_ref[...] = h * dis[:, None]


_lin_call = pl.pallas_call(
    _lin_body,
    grid=(N_NODES // BLK,),
    in_specs=[
        pl.BlockSpec((BLK, D), lambda i: (i, 0)),
        pl.BlockSpec((D, D), lambda i: (0, 0)),
        pl.BlockSpec((1, D), lambda i: (0, 0)),
        pl.BlockSpec((BLK, NC), lambda i: (i, 0)),
    ],
    out_specs=pl.BlockSpec((BLK, D), lambda i: (i, 0)),
    out_shape=jax.ShapeDtypeStruct((N_NODES, D), jnp.float32),
)


def _out_body(accp_ref, degp_ref, bias_ref, out_ref):
    acc = accp_ref[0] + accp_ref[1]
    deg = degp_ref[:, 0] + degp_ref[:, 1]
    dis = jnp.where(deg > 0.0, lax.rsqrt(deg), 0.0)
    out_ref[...] = jnp.maximum(acc * dis[:, None] + bias_ref[...], 0.0)


_out_call = pl.pallas_call(
    _out_body,
    grid=(N_NODES // BLK,),
    in_specs=[
        pl.BlockSpec((NC, BLK, D), lambda i: (0, i, 0)),
        pl.BlockSpec((BLK, NC), lambda i: (i, 0)),
        pl.BlockSpec((1, D), lambda i: (0, 0)),
    ],
    out_specs=pl.BlockSpec((BLK, D), lambda i: (i, 0)),
    out_shape=jax.ShapeDtypeStruct((N_NODES, D), jnp.float32),
)


@jax.jit
def kernel(x, edge_index, W, b_lin, bias):
    row = edge_index[0]
    col = edge_index[1]
    ones_k = jnp.ones((K,), jnp.float32)
    zeros_n = jnp.zeros((ZB,), jnp.float32)
    zrows = jnp.zeros((WB, D), jnp.float32)

    padcol = N_NODES + (jnp.arange(PAD, dtype=jnp.int32) % K)
    colp = jnp.concatenate(
        [col.reshape(NW, EPT), jnp.broadcast_to(padcol, (NW, PAD))], axis=1)
    col3 = colp.reshape(NW, NCH, K)

    degp = _deg_call(col3, ones_k, zeros_n)
    degp_t = degp.reshape(NC, N_NODES).T
    g = _lin_call(x, W, b_lin.reshape(1, D), degp_t)
    accp = _acc_call(g, row, col, zrows)
    out = _out_call(accp, degp_t, bias.reshape(1, D))
    return out
